# baseline (device time: 21723 ns/iter reference)
import jax
import jax.numpy as jnp
from jax import lax
from jax.experimental import pallas as pl
from jax.experimental.pallas import tpu as pltpu

N_DEV = 8
N_PEER = N_DEV - 1


def kernel(x):
    m, n = x.shape

    def body(x_ref, out_ref, send_buf, recv_buf, send_sems, recv_sems):
        my = lax.axis_index("i")

        def pair_rdma(slot, target):
            return pltpu.make_async_remote_copy(
                src_ref=send_buf,
                dst_ref=recv_buf.at[slot],
                send_sem=send_sems.at[slot],
                recv_sem=recv_sems.at[slot],
                device_id=(target,),
                device_id_type=pl.DeviceIdType.MESH,
            )

        barrier_sem = pltpu.get_barrier_semaphore()
        for r in range(N_DEV):
            @pl.when(my != r)
            def _():
                pl.semaphore_signal(
                    barrier_sem, inc=1,
                    device_id=(r,), device_id_type=pl.DeviceIdType.MESH,
                )
        pl.semaphore_wait(barrier_sem, N_DEV - 1)

        t = x_ref[:, :]
        rows = m
        while rows > 1:
            half = rows // 2
            t = t[:half, :] * t[half:rows, :]
            rows = half
        send_buf[:, :] = t

        for d in range(1, N_DEV):
            @pl.when(my + d < N_DEV)
            def _():
                pair_rdma(d - 1, my + d).start()

            @pl.when(my - d >= 0)
            def _():
                pair_rdma(N_PEER + d - 1, my - d).start()

        y = x_ref[:, :]
        d2 = 1
        while d2 < m:
            top = jnp.ones((d2, n), jnp.float32)
            y = y * jnp.concatenate([top, y[: m - d2, :]], axis=0)
            d2 *= 2

        for d in range(1, N_DEV):
            @pl.when(d <= my)
            def _():
                pair_rdma(d - 1, my - d).wait_recv()

            @pl.when(my + d < N_DEV)
            def _():
                pair_rdma(N_PEER + d - 1, my + d).wait_recv()

        prefix = jnp.ones((1, n), jnp.float32)
        for d in range(1, N_DEV):
            prefix = prefix * jnp.where(d <= my, recv_buf[d - 1], 1.0)

        out_ref[:, :] = y * prefix

        for d in range(1, N_DEV):
            @pl.when(my + d < N_DEV)
            def _():
                pair_rdma(d - 1, my + d).wait_send()

            @pl.when(my - d >= 0)
            def _():
                pair_rdma(N_PEER + d - 1, my - d).wait_send()

    return pl.pallas_call(
        body,
        out_shape=jax.ShapeDtypeStruct((m, n), jnp.float32),
        in_specs=[pl.BlockSpec(memory_space=pltpu.VMEM)],
        out_specs=pl.BlockSpec(memory_space=pltpu.VMEM),
        scratch_shapes=[
            pltpu.VMEM((1, n), jnp.float32),
            pltpu.VMEM((2 * N_PEER, 1, n), jnp.float32),
            pltpu.SemaphoreType.DMA((2 * N_PEER,)),
            pltpu.SemaphoreType.DMA((2 * N_PEER,)),
        ],
        compiler_params=pltpu.CompilerParams(collective_id=0),
    )(x)


# device time: 21213 ns/iter; 1.0240x vs baseline; 1.0240x over previous
import jax
import jax.numpy as jnp
from jax import lax
from jax.experimental import pallas as pl
from jax.experimental.pallas import tpu as pltpu

N_DEV = 8


def kernel(x):
    m, n = x.shape

    def body(x_ref, out_ref, send_buf, recv_buf, send_sems, recv_sems):
        my = lax.axis_index("i")

        def pair_rdma(slot, target):
            return pltpu.make_async_remote_copy(
                src_ref=send_buf,
                dst_ref=recv_buf.at[slot],
                send_sem=send_sems.at[slot],
                recv_sem=recv_sems.at[slot],
                device_id=(target,),
                device_id_type=pl.DeviceIdType.MESH,
            )

        barrier_sem = pltpu.get_barrier_semaphore()
        for r in range(N_DEV):
            @pl.when(my != r)
            def _():
                pl.semaphore_signal(
                    barrier_sem, inc=1,
                    device_id=(r,), device_id_type=pl.DeviceIdType.MESH,
                )
        pl.semaphore_wait(barrier_sem, N_DEV - 1)

        t = x_ref[:, :]
        rows = m
        while rows > 1:
            half = rows // 2
            t = t[:half, :] * t[half:rows, :]
            rows = half
        send_buf[:, :] = t

        for d in range(1, N_DEV):
            @pl.when(my + d < N_DEV)
            def _():
                pair_rdma(d - 1, my + d).start()

        y = x_ref[:, :]
        d2 = 1
        while d2 < m // 2:
            top = jnp.ones((d2, n), jnp.float32)
            y = y * jnp.concatenate([top, y[: m - d2, :]], axis=0)
            d2 *= 2

        for d in range(1, N_DEV):
            @pl.when(d <= my)
            def _():
                pair_rdma(d - 1, my - d).wait_recv()

        prefix = jnp.ones((1, n), jnp.float32)
        for d in range(1, N_DEV):
            prefix = prefix * jnp.where(d <= my, recv_buf[d - 1], 1.0)

        half = m // 2
        top = jnp.ones((half, n), jnp.float32)
        out_ref[:, :] = y * jnp.concatenate([top, y[:half, :]], axis=0) * prefix

        for d in range(1, N_DEV):
            @pl.when(my + d < N_DEV)
            def _():
                pair_rdma(d - 1, my + d).wait_send()

    return pl.pallas_call(
        body,
        out_shape=jax.ShapeDtypeStruct((m, n), jnp.float32),
        in_specs=[pl.BlockSpec(memory_space=pltpu.VMEM)],
        out_specs=pl.BlockSpec(memory_space=pltpu.VMEM),
        scratch_shapes=[
            pltpu.VMEM((1, n), jnp.float32),
            pltpu.VMEM((N_DEV - 1, 1, n), jnp.float32),
            pltpu.SemaphoreType.DMA((N_DEV - 1,)),
            pltpu.SemaphoreType.DMA((N_DEV - 1,)),
        ],
        compiler_params=pltpu.CompilerParams(collective_id=0),
    )(x)
